# Initial kernel scaffold; baseline (speedup 1.0000x reference)
import jax
import jax.numpy as jnp
from jax.experimental import pallas as pl


def kernel(x, Wc, bc, Wrc, brc, Wrg, brg, Wf, bf, Wfr, bfr):
    raise NotImplementedError("write your pallas kernel here")


# speedup vs baseline: 2.1328x; 2.1328x over previous
import functools

import jax
import jax.numpy as jnp
from jax.experimental import pallas as pl
from jax.experimental.pallas import tpu as pltpu


def _softmax(logits):
    m = jnp.max(logits, axis=-1, keepdims=True)
    e = jnp.exp(logits - m)
    return e / jnp.sum(e, axis=-1, keepdims=True)


def _conv_pool_kernel(p_ref, w_ref, q_ref, *, nb, ntile):
    y = jax.lax.dot_general(p_ref[...], w_ref[...],
                            dimension_numbers=(((0,), (0,)), ((), ())),
                            preferred_element_type=jnp.float32)
    y = jnp.maximum(y, 0.0)
    y = y.reshape(nb, 256, ntile).sum(axis=1)
    q_ref[...] = y.reshape(1, nb, ntile)


def _head_kernel(q_ref, wrg_ref, brg_ref, wfr_ref, bfr_ref, wfc_ref, bfo_ref,
                 out_ref, *, cl, e, nout):
    q0, q1, q2, q3 = q_ref[0], q_ref[1], q_ref[2], q_ref[3]
    qs = q0 + q1 + q2 + q3
    rfeat = qs[:, :cl] * (1.0 / 1024.0)
    logits = jnp.dot(rfeat, wrg_ref[...], preferred_element_type=jnp.float32)
    gates = _softmax(logits + brg_ref[...])
    hs = []
    for qd in (q0, q1, q2, q3):
        acc = gates[:, 0:1] * qd[:, cl:2 * cl]
        for ex in range(1, e):
            acc = acc + gates[:, ex:ex + 1] * qd[:, cl * (ex + 1):cl * (ex + 2)]
        hs.append(acc)
    h = jnp.concatenate(hs, axis=1) * (1.0 / 256.0)
    logits2 = jnp.dot(h, wfr_ref[...], preferred_element_type=jnp.float32)
    g2 = _softmax(logits2 + bfr_ref[...])
    z = jnp.dot(h, wfc_ref[...], preferred_element_type=jnp.float32)
    o = g2[:, 0:1] * z[:, 0:nout]
    for ex in range(1, e):
        o = o + g2[:, ex:ex + 1] * z[:, ex * nout:(ex + 1) * nout]
    o = o + jnp.dot(g2, bfo_ref[...], preferred_element_type=jnp.float32)
    out_ref[...] = o


def kernel(x, Wc, bc, Wrc, brc, Wrg, brg, Wf, bf, Wfr, bfr):
    B, Cin, H, W = x.shape
    E, Cl = Wc.shape[0], Wc.shape[1]
    Nout = Wf.shape[2]
    HW = H * W
    NB = 32
    SB = B // NB
    NCOLS = (1 + E) * Cl
    NTILE = 1152
    NT = NCOLS // NTILE
    K = Cin * 9 + 1
    QP = (H // 2) * (W // 2)

    Hq, Wq = H // 2, W // 2
    xp = jnp.pad(x, ((0, 0), (0, 0), (1, 1), (1, 1)))
    xq = jnp.stack([xp[:, :, Hq * qi:Hq * qi + Hq + 2, Wq * qj:Wq * qj + Wq + 2]
                    for qi in range(2) for qj in range(2)], axis=0)
    rows = [xq[:, :, ci, dy:dy + Hq, dx:dx + Wq].reshape(1, 4 * B * QP)
            for ci in range(Cin) for dy in range(3) for dx in range(3)]
    rows.append(jnp.ones((1, 4 * B * QP), x.dtype))
    P = jnp.concatenate(rows, axis=0).astype(jnp.bfloat16)

    Wmat = jnp.concatenate([Wrc.reshape(Cl, Cin * 9),
                            Wc.reshape(E * Cl, Cin * 9)], axis=0).T
    brow = jnp.concatenate([brc, bc.reshape(E * Cl)])[None, :]
    Wall = jnp.concatenate([Wmat, brow], axis=0).astype(jnp.bfloat16)

    q = pl.pallas_call(
        functools.partial(_conv_pool_kernel, nb=NB, ntile=NTILE),
        grid=(4, SB, NT),
        in_specs=[
            pl.BlockSpec((K, NB * QP), lambda qd, sb, nt: (0, qd * (B // NB) + sb)),
            pl.BlockSpec((K, NTILE), lambda qd, sb, nt: (0, nt)),
        ],
        out_specs=pl.BlockSpec((1, NB, NTILE), lambda qd, sb, nt: (qd, sb, nt)),
        out_shape=jax.ShapeDtypeStruct((4, B, NCOLS), jnp.float32),
    )(P, Wall)

    Wfr_p = Wfr.reshape(Cl, 4, E).transpose(1, 0, 2).reshape(4 * Cl, E)
    Wf_p = Wf.reshape(E, Cl, 4, Nout).transpose(0, 2, 1, 3)
    Wf_cat = Wf_p.reshape(E, 4 * Cl, Nout).transpose(1, 0, 2).reshape(4 * Cl, E * Nout)

    BS = min(128, B)
    out = pl.pallas_call(
        functools.partial(_head_kernel, cl=Cl, e=E, nout=Nout),
        grid=(B // BS,),
        in_specs=[
            pl.BlockSpec((4, BS, NCOLS), lambda b: (0, b, 0)),
            pl.BlockSpec((Cl, E), lambda b: (0, 0)),
            pl.BlockSpec((1, E), lambda b: (0, 0)),
            pl.BlockSpec((4 * Cl, E), lambda b: (0, 0)),
            pl.BlockSpec((1, E), lambda b: (0, 0)),
            pl.BlockSpec((4 * Cl, E * Nout), lambda b: (0, 0)),
            pl.BlockSpec((E, Nout), lambda b: (0, 0)),
        ],
        out_specs=pl.BlockSpec((BS, Nout), lambda b: (b, 0)),
        out_shape=jax.ShapeDtypeStruct((B, Nout), jnp.float32),
    )(q, Wrg, brg.reshape(1, E), Wfr_p, bfr.reshape(1, E), Wf_cat, bf)
    return out


# speedup vs baseline: 2.7746x; 1.3009x over previous
import functools

import jax
import jax.numpy as jnp
from jax.experimental import pallas as pl
from jax.experimental.pallas import tpu as pltpu


def _softmax(logits):
    m = jnp.max(logits, axis=-1, keepdims=True)
    e = jnp.exp(logits - m)
    return e / jnp.sum(e, axis=-1, keepdims=True)


def _conv_pool_kernel(p_ref, w_ref, q_ref, *, nb, ntile):
    y = jax.lax.dot_general(p_ref[...], w_ref[...],
                            dimension_numbers=(((0,), (0,)), ((), ())),
                            preferred_element_type=jnp.float32)
    y = jnp.maximum(y, 0.0)
    q_ref[...] = y.reshape(nb * 4, 256, ntile).sum(axis=1)


def _head_kernel(q_ref, tsum_ref, t4_ref, wrg_ref, brg_ref, wfr_ref, bfr_ref,
                 wfc_ref, bfo_ref, out_ref, *, cl, e, nout, bs):
    qb = q_ref[...]
    rfeat = jnp.dot(tsum_ref[...], qb[:, :cl],
                    preferred_element_type=jnp.float32) * (1.0 / 1024.0)
    logits = jnp.dot(rfeat, wrg_ref[...], preferred_element_type=jnp.float32)
    gates = _softmax(logits + brg_ref[...])
    grows = jax.lax.dot_general(tsum_ref[...], gates,
                                dimension_numbers=(((0,), (0,)), ((), ())),
                                preferred_element_type=jnp.float32)
    acc = grows[:, 0:1] * qb[:, cl:2 * cl]
    for ex in range(1, e):
        acc = acc + grows[:, ex:ex + 1] * qb[:, cl * (ex + 1):cl * (ex + 2)]
    h4 = jnp.dot(t4_ref[...], acc, preferred_element_type=jnp.float32)
    h = jnp.concatenate([h4[qd * bs:(qd + 1) * bs] for qd in range(4)],
                        axis=1) * (1.0 / 256.0)
    logits2 = jnp.dot(h, wfr_ref[...], preferred_element_type=jnp.float32)
    g2 = _softmax(logits2 + bfr_ref[...])
    z = jnp.dot(h, wfc_ref[...], preferred_element_type=jnp.float32)
    o = g2[:, 0:1] * z[:, 0:nout]
    for ex in range(1, e):
        o = o + g2[:, ex:ex + 1] * z[:, ex * nout:(ex + 1) * nout]
    o = o + jnp.dot(g2, bfo_ref[...], preferred_element_type=jnp.float32)
    out_ref[...] = o


def kernel(x, Wc, bc, Wrc, brc, Wrg, brg, Wf, bf, Wfr, bfr):
    B, Cin, H, W = x.shape
    E, Cl = Wc.shape[0], Wc.shape[1]
    Nout = Wf.shape[2]
    HW = H * W
    NB = 8
    SB = B // NB
    NCOLS = (1 + E) * Cl
    NTILE = 1152
    NT = NCOLS // NTILE
    K = Cin * 9 + 1

    Hq, Wq = H // 2, W // 2
    xa = jnp.concatenate([x, jnp.ones((B, 1, H, W), x.dtype)], axis=1)
    xp = jnp.pad(xa, ((0, 0), (0, 0), (1, 1), (1, 1))).astype(jnp.bfloat16)
    xqp = jnp.stack([xp[:, :, Hq * qi:Hq * qi + Hq + 2, Wq * qj:Wq * qj + Wq + 2]
                     for qi in range(2) for qj in range(2)], axis=2)
    taps = [xqp[:, ci, :, dy:dy + Hq, dx:dx + Wq]
            for ci in range(Cin) for dy in range(3) for dx in range(3)]
    taps.append(xqp[:, Cin, :, 1:1 + Hq, 1:1 + Wq])
    P = jnp.stack(taps, axis=0)
    P = P.reshape(K, B * HW)

    Wmat = jnp.concatenate([Wrc.reshape(Cl, Cin * 9),
                            Wc.reshape(E * Cl, Cin * 9)], axis=0).T
    brow = jnp.concatenate([brc, bc.reshape(E * Cl)])[None, :]
    Wall = jnp.concatenate([Wmat, brow], axis=0).astype(jnp.bfloat16)

    q = pl.pallas_call(
        functools.partial(_conv_pool_kernel, nb=NB, ntile=NTILE),
        grid=(SB, NT),
        in_specs=[
            pl.BlockSpec((K, NB * HW), lambda sb, nt: (0, sb)),
            pl.BlockSpec((K, NTILE), lambda sb, nt: (0, nt)),
        ],
        out_specs=pl.BlockSpec((NB * 4, NTILE), lambda sb, nt: (sb, nt)),
        out_shape=jax.ShapeDtypeStruct((B * 4, NCOLS), jnp.float32),
    )(P, Wall)

    Wfr_p = Wfr.reshape(Cl, 4, E).transpose(1, 0, 2).reshape(4 * Cl, E)
    Wf_p = Wf.reshape(E, Cl, 4, Nout).transpose(0, 2, 1, 3)
    Wf_cat = Wf_p.reshape(E, 4 * Cl, Nout).transpose(1, 0, 2).reshape(4 * Cl, E * Nout)

    BS = min(128, B)
    sidx = jnp.arange(BS * 4)
    Tsum = (jnp.arange(BS)[:, None] == (sidx[None, :] // 4)).astype(jnp.float32)
    T4 = (sidx[None, :] == (4 * (sidx[:, None] % BS) + sidx[:, None] // BS)
          ).astype(jnp.float32)
    out = pl.pallas_call(
        functools.partial(_head_kernel, cl=Cl, e=E, nout=Nout, bs=BS),
        grid=(B // BS,),
        in_specs=[
            pl.BlockSpec((BS * 4, NCOLS), lambda b: (b, 0)),
            pl.BlockSpec((BS, BS * 4), lambda b: (0, 0)),
            pl.BlockSpec((BS * 4, BS * 4), lambda b: (0, 0)),
            pl.BlockSpec((Cl, E), lambda b: (0, 0)),
            pl.BlockSpec((1, E), lambda b: (0, 0)),
            pl.BlockSpec((4 * Cl, E), lambda b: (0, 0)),
            pl.BlockSpec((1, E), lambda b: (0, 0)),
            pl.BlockSpec((4 * Cl, E * Nout), lambda b: (0, 0)),
            pl.BlockSpec((E, Nout), lambda b: (0, 0)),
        ],
        out_specs=pl.BlockSpec((BS, Nout), lambda b: (b, 0)),
        out_shape=jax.ShapeDtypeStruct((B, Nout), jnp.float32),
    )(q, Tsum, T4, Wrg, brg.reshape(1, E), Wfr_p, bfr.reshape(1, E), Wf_cat, bf)
    return out
